# baseline (speedup 1.0000x reference)
import functools

import jax
import jax.numpy as jnp
from jax import lax
from jax.experimental import pallas as pl
from jax.experimental.pallas import tpu as pltpu, tpu_sc as plsc

D = 64
B = 4096 * 200
SCALE = 8.0

_info = plsc.get_sparse_core_info()
NC, NS, L = _info.num_cores, _info.num_subcores, _info.num_lanes
NW = NC * NS
B_PER_W = B // NW
CB = 160
NBUF = 2
N_CHUNKS = B_PER_W // CB


def _sc_embed(x_flat, lut_pairs):
    mesh = plsc.VectorSubcoreMesh(core_axis_name="c", subcore_axis_name="s")

    @functools.partial(
        pl.kernel,
        mesh=mesh,
        compiler_params=pltpu.CompilerParams(use_tc_tiling_on_sc=True,
                                             needs_layout_passes=False),
        out_type=jax.ShapeDtypeStruct((B, D), jnp.float32),
        scratch_types=(
            [pltpu.VMEM((B_PER_W,), jnp.int32)]
            + [pltpu.VMEM((CB,), jnp.int32) for _ in range(NBUF)]
            + [pltpu.VMEM((CB, 2 * D), jnp.float32) for _ in range(NBUF)]
            + [pltpu.VMEM((CB, D), jnp.float32) for _ in range(NBUF)]
            + [pltpu.SemaphoreType.DMA for _ in range(2 * NBUF)]
        ),
    )
    def k(idx_hbm, table_hbm, out_hbm, idx_all, *bufs_and_sems):
        pidx = bufs_and_sems[:NBUF]
        pair = bufs_and_sems[NBUF:2 * NBUF]
        outb = bufs_and_sems[2 * NBUF:3 * NBUF]
        sg = bufs_and_sems[3 * NBUF:4 * NBUF]
        sw = bufs_and_sems[4 * NBUF:5 * NBUF]

        wid = lax.axis_index("s") * NC + lax.axis_index("c")
        base = wid * B_PER_W
        pltpu.sync_copy(idx_hbm.at[pl.ds(base, B_PER_W)], idx_all)

        lanes = lax.iota(jnp.int32, L)

        def prep_pidx(g, b):
            def body(j, c):
                sl = pl.ds(g * CB + j * L, L)
                pidx[b][pl.ds(j * L, L)] = lax.shift_right_logical(
                    idx_all[sl], 1)
                return c
            lax.fori_loop(0, CB // L, body, 0, unroll=4)

        def gather_start(b):
            pltpu.async_copy(table_hbm.at[pidx[b]], pair[b], sg[b])

        def gather_wait(b):
            pltpu.make_async_copy(table_hbm.at[pidx[b]], pair[b],
                                  sg[b]).wait()

        def wb_start(g, b):
            pltpu.async_copy(outb[b],
                             out_hbm.at[pl.ds(base + g * CB, CB)], sw[b])

        def wb_wait(b):
            pltpu.make_async_copy(outb[b],
                                  out_hbm.at[pl.ds(base, CB)], sw[b]).wait()

        def select_scale(g, b):
            def group(j, c):
                rows = j * L + lanes
                hcol = (idx_all[pl.ds(g * CB + j * L, L)] & 1) * D

                def col(ci, c2):
                    v = plsc.load_gather(pair[b], [rows, hcol + ci])
                    plsc.store_scatter(outb[b],
                                       [rows, jnp.full((L,), ci, jnp.int32)],
                                       v * SCALE)
                    return c2
                lax.fori_loop(0, D, col, 0, unroll=8)
                return c
            lax.fori_loop(0, CB // L, group, 0)

        prep_pidx(0, 0)
        gather_start(0)

        def outer(go, carry):
            for b in range(NBUF):
                g = go * NBUF + b
                nb = (b + 1) % NBUF

                @pl.when(g + 1 < N_CHUNKS)
                def _():
                    prep_pidx(g + 1, nb)

                @pl.when(jnp.logical_and(g + 1 < N_CHUNKS, g >= 1))
                def _():
                    wb_wait(nb)

                @pl.when(g + 1 < N_CHUNKS)
                def _():
                    gather_start(nb)

                gather_wait(b)
                select_scale(g, b)
                wb_start(g, b)
            return carry

        lax.fori_loop(0, N_CHUNKS // NBUF, outer, 0)
        for b in range(NBUF):
            wb_wait(b)

    return k(x_flat, lut_pairs)


def kernel(x, lut):
    x_flat = x.reshape(-1).astype(jnp.int32)
    lut_pairs = lut.reshape(lut.shape[0] // 2, 2 * D)
    out = _sc_embed(x_flat, lut_pairs)
    return out.reshape(x.shape[0], x.shape[1], D)


# speedup vs baseline: 2.0400x; 2.0400x over previous
import functools

import jax
import jax.numpy as jnp
from jax import lax
from jax.experimental import pallas as pl
from jax.experimental.pallas import tpu as pltpu, tpu_sc as plsc

D = 64
R = 4096
C = 200
SCALE = 8.0

_info = plsc.get_sparse_core_info()
NC, NS, L = _info.num_cores, _info.num_subcores, _info.num_lanes
NW = NC * NS
R_PER_W = R // NW
NBUF = 2


def _sc_embed(x_flat, lut_wide):
    mesh = plsc.VectorSubcoreMesh(core_axis_name="c", subcore_axis_name="s")

    @functools.partial(
        pl.kernel,
        mesh=mesh,
        compiler_params=pltpu.CompilerParams(use_tc_tiling_on_sc=True,
                                             needs_layout_passes=False),
        out_type=jax.ShapeDtypeStruct((R, C, D), jnp.float32),
        scratch_types=(
            [pltpu.VMEM((C,), jnp.int32) for _ in range(NBUF)]
            + [pltpu.VMEM((C, 2 * D), jnp.float32) for _ in range(NBUF)]
            + [pltpu.VMEM((1, C, D), jnp.float32) for _ in range(NBUF)]
            + [pltpu.SemaphoreType.DMA for _ in range(3 * NBUF)]
        ),
    )
    def k(idx_hbm, table_hbm, out_hbm, *bufs_and_sems):
        idxb = bufs_and_sems[:NBUF]
        pair = bufs_and_sems[NBUF:2 * NBUF]
        outb = bufs_and_sems[2 * NBUF:3 * NBUF]
        si = bufs_and_sems[3 * NBUF:4 * NBUF]
        sg = bufs_and_sems[4 * NBUF:5 * NBUF]
        sw = bufs_and_sems[5 * NBUF:6 * NBUF]

        wid = lax.axis_index("s") * NC + lax.axis_index("c")
        base = wid * R_PER_W

        def idx_start(g, b):
            pltpu.async_copy(idx_hbm.at[pl.ds((base + g) * C, C)], idxb[b],
                             si[b])

        def idx_wait(b):
            pltpu.make_async_copy(idx_hbm.at[pl.ds(base * C, C)], idxb[b],
                                  si[b]).wait()

        def gather_start(b):
            pltpu.async_copy(table_hbm.at[idxb[b]], pair[b], sg[b])

        def gather_wait(b):
            pltpu.make_async_copy(table_hbm.at[idxb[b]], pair[b],
                                  sg[b]).wait()

        def wb_start(g, b):
            pltpu.async_copy(outb[b], out_hbm.at[pl.ds(base + g, 1)], sw[b])

        def wb_wait(b):
            pltpu.make_async_copy(outb[b], out_hbm.at[pl.ds(base, 1)],
                                  sw[b]).wait()

        def scale(b):
            def row(r, c):
                for k4 in range(D // L):
                    sl = pl.ds(k4 * L, L)
                    outb[b][0, r, sl] = pair[b][r, sl] * SCALE
                return c
            lax.fori_loop(0, C, row, 0, unroll=8)

        idx_start(0, 0)
        idx_start(1, 1)
        idx_wait(0)
        gather_start(0)

        def outer(go, carry):
            for b in range(NBUF):
                g = go * NBUF + b
                nb = (b + 1) % NBUF

                gather_wait(b)

                @pl.when(g + 2 < R_PER_W)
                def _():
                    idx_start(g + 2, b)

                @pl.when(jnp.logical_and(g + 1 < R_PER_W, g >= 1))
                def _():
                    wb_wait(nb)

                @pl.when(g + 1 < R_PER_W)
                def _():
                    idx_wait(nb)
                    gather_start(nb)

                scale(b)
                wb_start(g, b)
            return carry

        lax.fori_loop(0, R_PER_W // NBUF, outer, 0)
        for b in range(NBUF):
            wb_wait(b)

    return k(x_flat, lut_wide)


def kernel(x, lut):
    x_flat = x.reshape(-1).astype(jnp.int32)
    lut_wide = jnp.pad(lut, ((0, 0), (0, D)))
    return _sc_embed(x_flat, lut_wide)
